# baseline (speedup 1.0000x reference)
import jax
import jax.numpy as jnp
from jax import lax
from jax.experimental import pallas as pl
from jax.experimental.pallas import tpu as pltpu
from jax.experimental.pallas import tpu_sc as plsc

_B, _C, _H, _W = 8, 255, 76, 76
_S = _H * _W
_COLS = 4 * _W
_HALF = _COLS // 2
_NT = _S // _COLS
_PW = (10.0 / 608.0, 16.0 / 608.0, 33.0 / 608.0)
_PH = (13.0 / 608.0, 30.0 / 608.0, 23.0 / 608.0)
_OFFS0 = tuple(range(0, 160, 16))
_OFFS1 = tuple(range(152, 296, 16)) + (288,)


def _decode_body(x_ref, y_ref, inb, idxb, outb, sem):
    wid = lax.axis_index("s") * 2 + lax.axis_index("c")
    b = wid // 4
    q = wid % 4
    start = jnp.where(q < 3, q * 5, 15)
    trip = jnp.where(q < 3, 5, 4)
    iota = lax.iota(jnp.int32, 16)
    viota = iota * _C
    mask8 = iota < 8

    def tile(k, carry):
        j = start + k
        row0 = 4 * j
        base = b * (_C * _NT) + j
        for t in range(16):
            coff = jnp.minimum(iota + 16 * t, _C - 1) * _NT
            idxb[pl.ds(16 * t, 16)] = coff + base
        pltpu.async_copy(x_ref.at[idxb], inb, sem).wait()

        for hh, offs in ((0, _OFFS0), (1, _OFFS1)):
            for off in offs:
                msk = mask8 if off == 144 else None
                scv = off + iota
                tsum = ((scv >= _W).astype(jnp.int32)
                        + (scv >= 2 * _W).astype(jnp.int32)
                        + (scv >= 3 * _W).astype(jnp.int32))
                wvf = (scv - _W * tsum).astype(jnp.float32)
                hvf = (jnp.full((16,), row0, jnp.int32)
                       + tsum).astype(jnp.float32)
                vx = [inb[85 * a + 0, pl.ds(off, 16)] for a in range(3)]
                vy = [inb[85 * a + 1, pl.ds(off, 16)] for a in range(3)]
                vw = [inb[85 * a + 2, pl.ds(off, 16)] for a in range(3)]
                vh = [inb[85 * a + 3, pl.ds(off, 16)] for a in range(3)]
                sx = [1.0 / (1.0 + jnp.exp(-v)) for v in vx]
                sy = [1.0 / (1.0 + jnp.exp(-v)) for v in vy]
                ew = [jnp.exp(v) for v in vw]
                eh = [jnp.exp(v) for v in vh]
                rx = [(s + wvf) * (1.0 / _W) for s in sx]
                ry = [(s + hvf) * (1.0 / _H) for s in sy]
                rw = [_PW[a] * ew[a] for a in range(3)]
                rh = [_PH[a] * eh[a] for a in range(3)]
                for a in range(3):
                    bb = (off - hh * _HALF) * _C + 85 * a
                    plsc.store_scatter(outb, [viota + bb], rx[a], mask=msk)
                    plsc.store_scatter(outb, [viota + (bb + 1)], ry[a],
                                       mask=msk)
                    plsc.store_scatter(outb, [viota + (bb + 2)], rw[a],
                                       mask=msk)
                    plsc.store_scatter(outb, [viota + (bb + 3)], rh[a],
                                       mask=msk)

            @plsc.parallel_loop(0, 81, 1, unroll=1)
            def ch(i):
                for a in range(3):
                    c = 85 * a + 4 + i
                    vs = [inb[c, pl.ds(off, 16)] for off in offs]
                    rs = [1.0 / (1.0 + jnp.exp(-v)) for v in vs]
                    for off, r in zip(offs, rs):
                        msk = mask8 if off == 144 else None
                        plsc.store_scatter(
                            outb, [viota + ((off - hh * _HALF) * _C + c)],
                            r, mask=msk)

            pltpu.sync_copy(
                outb,
                y_ref.at[b, pl.ds((j * _COLS + hh * _HALF) * _C,
                                  _HALF * _C)],
            )
        return carry

    lax.fori_loop(0, trip, tile, 0)


def kernel(x):
    xr = x.reshape(_B * _C * _NT, _COLS)
    mesh = plsc.VectorSubcoreMesh(core_axis_name="c", subcore_axis_name="s")
    y = pl.kernel(
        _decode_body,
        out_type=jax.ShapeDtypeStruct((_B, _S * _C), jnp.float32),
        mesh=mesh,
        scratch_types=[
            pltpu.VMEM((256, _COLS), jnp.float32),
            pltpu.VMEM((256,), jnp.int32),
            pltpu.VMEM((_HALF * _C,), jnp.float32),
            pltpu.SemaphoreType.DMA,
        ],
        compiler_params=pltpu.CompilerParams(
            use_tc_tiling_on_sc=False, needs_layout_passes=False),
    )(xr)
    return y.reshape(_B, _S * _C // 85, 85)


# speedup vs baseline: 1.0003x; 1.0003x over previous
import jax
import jax.numpy as jnp
from jax import lax
from jax.experimental import pallas as pl
from jax.experimental.pallas import tpu as pltpu
from jax.experimental.pallas import tpu_sc as plsc

_B, _C, _H, _W = 8, 255, 76, 76
_S = _H * _W
_NC, _NS = 2, 16
_COLS = 2 * _W
_NT = _S // _COLS
_PW = (10.0 / 608.0, 16.0 / 608.0, 33.0 / 608.0)
_PH = (13.0 / 608.0, 30.0 / 608.0, 23.0 / 608.0)
_OFFS = (0, 16, 32, 48, 64, 80, 96, 112, 128, 136)


def _decode_body(x_ref, y_ref, inb, idxb, outb, sem):
    wid = lax.axis_index("s") * _NC + lax.axis_index("c")
    b = wid // 4
    q = wid % 4
    start = jnp.where(q < 2, q * 10, 20 + (q - 2) * 9)
    trip = jnp.where(q < 2, 10, 9)
    iota = lax.iota(jnp.int32, 16)

    viota = iota * _C

    def tile(k, carry):
        j = start + k
        s0 = j * _COLS
        row0 = 2 * j
        base = b * (_C * _NT) + j
        for t in range(16):
            coff = jnp.minimum(iota + 16 * t, _C - 1) * _NT
            idxb[pl.ds(16 * t, 16)] = coff + base
        pltpu.async_copy(x_ref.at[idxb], inb, sem).wait()

        for off in _OFFS:
            scv = off + iota
            ge = scv >= _W
            wvf = jnp.where(ge, scv - _W, scv).astype(jnp.float32)
            hvf = (jnp.full((16,), row0, jnp.int32)
                   + ge.astype(jnp.int32)).astype(jnp.float32)
            vx = [inb[85 * a + 0, pl.ds(off, 16)] for a in range(3)]
            vy = [inb[85 * a + 1, pl.ds(off, 16)] for a in range(3)]
            vw = [inb[85 * a + 2, pl.ds(off, 16)] for a in range(3)]
            vh = [inb[85 * a + 3, pl.ds(off, 16)] for a in range(3)]
            sx = [1.0 / (1.0 + jnp.exp(-v)) for v in vx]
            sy = [1.0 / (1.0 + jnp.exp(-v)) for v in vy]
            ew = [jnp.exp(v) for v in vw]
            eh = [jnp.exp(v) for v in vh]
            rx = [(s + wvf) * (1.0 / _W) for s in sx]
            ry = [(s + hvf) * (1.0 / _H) for s in sy]
            rw = [_PW[a] * ew[a] for a in range(3)]
            rh = [_PH[a] * eh[a] for a in range(3)]
            for a in range(3):
                base = off * _C + 85 * a
                plsc.store_scatter(outb, [viota + base], rx[a])
                plsc.store_scatter(outb, [viota + (base + 1)], ry[a])
                plsc.store_scatter(outb, [viota + (base + 2)], rw[a])
                plsc.store_scatter(outb, [viota + (base + 3)], rh[a])

        @plsc.parallel_loop(0, 81, 1, unroll=1)
        def ch(i):
            for a in range(3):
                c = 85 * a + 4 + i
                vs = [inb[c, pl.ds(off, 16)] for off in _OFFS]
                rs = [1.0 / (1.0 + jnp.exp(-v)) for v in vs]
                for off, r in zip(_OFFS, rs):
                    plsc.store_scatter(outb, [viota + (off * _C + c)], r)

        pltpu.sync_copy(outb, y_ref.at[b, pl.ds(s0 * _C, _COLS * _C)])
        return carry

    lax.fori_loop(0, trip, tile, 0)


def kernel(x):
    xr = x.reshape(_B * _C * _NT, _COLS)
    mesh = plsc.VectorSubcoreMesh(core_axis_name="c", subcore_axis_name="s")
    y = pl.kernel(
        _decode_body,
        out_type=jax.ShapeDtypeStruct((_B, _S * _C), jnp.float32),
        mesh=mesh,
        scratch_types=[
            pltpu.VMEM((256, _COLS), jnp.float32),
            pltpu.VMEM((256,), jnp.int32),
            pltpu.VMEM((_COLS * _C,), jnp.float32),
            pltpu.SemaphoreType.DMA,
        ],
        compiler_params=pltpu.CompilerParams(
            use_tc_tiling_on_sc=False, needs_layout_passes=False),
    )(xr)
    return y.reshape(_B, _S * _C // 85, 85)
